# baseline (speedup 1.0000x reference)
import functools

import jax
import jax.numpy as jnp
from jax import lax
from jax.experimental import pallas as pl
from jax.experimental.pallas import tpu as pltpu
from jax.experimental.pallas import tpu_sc as plsc

NC = 2
NS = 16
L = 16
NW = NC * NS
C = 320
MARGIN = 0.2
QS = 1024.0

_GDN = lax.GatherDimensionNumbers(
    offset_dims=(), collapsed_slice_dims=(0,), start_index_map=(0,))


def _lanesum(v, lane):
  for sh in (8, 4, 2, 1):
    perm = lane ^ sh
    v = v + lax.gather(v, perm[:, None], _GDN, (1,),
                       mode=lax.GatherScatterMode.PROMISE_IN_BOUNDS)
  return v


def _halves(v):
  lo = v & jnp.int32(0xFFFF)
  hi = lax.shift_right_logical(v, 16)
  return lo, hi


def _sc_body(n, ch0, ch1, dw, table, ia, ip, inn, out,
             idx0, idx1, rows0, rows1, acc_v, si0, si1, sr0, sr1):
  idxb = (idx0, idx1)
  rowsb = (rows0, rows1)
  sib = (si0, si1)
  srb = (sr0, sr1)
  isrc = (ia, ip, inn)
  c = lax.axis_index("c")
  s = lax.axis_index("s")
  wid = s * NC + c
  ch = jnp.where(c == 0, ch0, ch1)
  wcbase = jnp.where(c == 0, s * ch0, NS * ch0 + s * ch1)
  lane = lax.iota(jnp.int32, 16)
  zero = jnp.zeros((L,), jnp.float32)

  def idx_start(k, b):
    base = (wcbase + k) * C
    for j in range(3):
      pltpu.async_copy(isrc[j].at[pl.ds(base, C)],
                       idxb[b].at[pl.ds(j * C, C)], sib[b])

  def idx_wait(b):
    for j in range(3):
      pltpu.make_async_copy(isrc[j].at[pl.ds(0, C)],
                            idxb[b].at[pl.ds(j * C, C)], sib[b]).wait()

  def g_start(b):
    pltpu.async_copy(table.at[idxb[b]], rowsb[b], srb[b])

  def g_wait(b):
    pltpu.make_async_copy(table.at[idxb[b]], rowsb[b], srb[b]).wait()

  def compute(k, rows, total):
    base = (wcbase + k) * C

    def tbody(t, acc):
      dap = zero
      dan = zero
      for kk in range(dw // L):
        av = rows[t, pl.ds(kk * L, L)]
        pv = rows[C + t, pl.ds(kk * L, L)]
        nv = rows[2 * C + t, pl.ds(kk * L, L)]
        alo, ahi = _halves(av)
        plo, phi = _halves(pv)
        nlo, nhi = _halves(nv)
        dpl = (alo - plo).astype(jnp.float32)
        dph = (ahi - phi).astype(jnp.float32)
        dnl = (alo - nlo).astype(jnp.float32)
        dnh = (ahi - nhi).astype(jnp.float32)
        dap = dap + dpl * dpl + dph * dph
        dan = dan + dnl * dnl + dnh * dnh
      r = (dap - dan) * (1.0 / (QS * QS))
      r = jnp.where(base + t < n, r, -1e30)
      s2 = _lanesum(r, lane)
      ls = jnp.maximum(s2 + MARGIN, 0.0)
      return acc + ls

    return lax.fori_loop(0, C, tbody, total, unroll=2)

  idx_start(0, 0)
  idx_start(1, 1)
  idx_wait(0)
  g_start(0)

  def outer(k2, total):
    for b in range(2):
      k = 2 * k2 + b
      nb = 1 - b
      g_wait(b)

      @pl.when(k + 2 < ch)
      def _():
        idx_start(k + 2, b)

      @pl.when(k + 1 < ch)
      def _():
        idx_wait(nb)
        g_start(nb)

      total = compute(k, rowsb[b], total)
    return total

  total = lax.fori_loop(0, ch // 2, outer, zero)
  acc_v[...] = total
  pltpu.sync_copy(acc_v, out.at[pl.ds(wid * L, L)])


def _reduce_body(n, x_ref, o_ref):
  o_ref[...] = (jnp.sum(x_ref[...]) / (L * n)).reshape(1, 1)


def kernel(fg_embed, triplet_index):
  n = triplet_index.shape[1]
  d = fg_embed.shape[1]
  dw = d // 2
  ch = -(-n // (NW * C))
  ch = ch + (ch % 2)
  ch1 = (int(2 * ch * 0.39) // 2) * 2
  ch0 = 2 * ch - ch1
  npad = NS * (ch0 + ch1) * C

  idx = triplet_index.astype(jnp.int32)
  if npad > n:
    idx = jnp.pad(idx, ((0, 0), (0, npad - n)))
  q = jnp.clip(jnp.round(fg_embed * QS) + 32768.0, 0.0, 65535.0)
  q = q.astype(jnp.int32).reshape(-1, dw, 2)
  packed = jnp.bitwise_or(q[..., 0], jnp.left_shift(q[..., 1], 16))

  mesh = plsc.VectorSubcoreMesh(core_axis_name="c", subcore_axis_name="s",
                                num_cores=NC, num_subcores=NS)
  sc_call = pl.kernel(
      functools.partial(_sc_body, n, ch0, ch1, dw),
      out_type=jax.ShapeDtypeStruct((NW * L,), jnp.float32),
      mesh=mesh,
      compiler_params=pltpu.CompilerParams(use_tc_tiling_on_sc=False),
      scratch_types=[
          pltpu.VMEM((3 * C,), jnp.int32),
          pltpu.VMEM((3 * C,), jnp.int32),
          pltpu.VMEM((3 * C, dw), jnp.int32),
          pltpu.VMEM((3 * C, dw), jnp.int32),
          pltpu.VMEM((L,), jnp.float32),
          pltpu.SemaphoreType.DMA,
          pltpu.SemaphoreType.DMA,
          pltpu.SemaphoreType.DMA,
          pltpu.SemaphoreType.DMA,
      ],
  )
  partials = sc_call(packed, idx[0], idx[1], idx[2])

  loss = pl.pallas_call(
      functools.partial(_reduce_body, n),
      out_shape=jax.ShapeDtypeStruct((1, 1), jnp.float32),
  )(partials.reshape(4, NW * L // 4))
  return loss[0, 0]


# speedup vs baseline: 2.7310x; 2.7310x over previous
import functools

import jax
import jax.numpy as jnp
from jax import lax
from jax.experimental import pallas as pl
from jax.experimental.pallas import tpu as pltpu
from jax.experimental.pallas import tpu_sc as plsc

NC = 2
NS = 16
L = 16
NW = NC * NS
C = 160
MARGIN = 0.2

_GDN = lax.GatherDimensionNumbers(
    offset_dims=(), collapsed_slice_dims=(0,), start_index_map=(0,))


def _lanesum(v, lane):
  for sh in (8, 4, 2, 1):
    perm = lane ^ sh
    v = v + lax.gather(v, perm[:, None], _GDN, (1,),
                       mode=lax.GatherScatterMode.PROMISE_IN_BOUNDS)
  return v


def _sc_body(n, ch0, ch1, table, ia, ip, inn, out,
             i00, i01, i02, i10, i11, i12,
             r00, r01, r02, r10, r11, r12,
             acc_v, si0, si1, sr0, sr1):
  ib = ((i00, i01, i02), (i10, i11, i12))
  rb = ((r00, r01, r02), (r10, r11, r12))
  sib = (si0, si1)
  srb = (sr0, sr1)
  isrc = (ia, ip, inn)
  c = lax.axis_index("c")
  s = lax.axis_index("s")
  wid = s * NC + c
  ch = jnp.where(c == 0, ch0, ch1)
  wcbase = jnp.where(c == 0, s * ch0, NS * ch0 + s * ch1)
  lane = lax.iota(jnp.int32, 16)
  zero = jnp.zeros((L,), jnp.float32)

  def idx_start(k, b):
    base = (wcbase + k) * C
    for j in range(3):
      pltpu.async_copy(isrc[j].at[pl.ds(base, C)], ib[b][j], sib[b])

  def idx_wait(b):
    for j in range(3):
      pltpu.make_async_copy(isrc[j].at[pl.ds(0, C)], ib[b][j], sib[b]).wait()

  def g_start(b):
    for j in range(3):
      pltpu.async_copy(table.at[ib[b][j]], rb[b][j], srb[b])

  def g_wait(b):
    for j in range(3):
      pltpu.make_async_copy(table.at[ib[b][j]], rb[b][j], srb[b]).wait()

  def compute(k, b, total):
    base = (wcbase + k) * C
    ra, rp, rn = rb[b]
    d = table.shape[1]

    def tbody(t, acc):
      dap = zero
      dan = zero
      for kk in range(d // L):
        av = ra[t, pl.ds(kk * L, L)]
        pv = rp[t, pl.ds(kk * L, L)]
        nv = rn[t, pl.ds(kk * L, L)]
        dp = av - pv
        dn = av - nv
        dap = dap + dp * dp
        dan = dan + dn * dn
      r = dap - dan
      r = jnp.where(base + t < n, r, -1e30)
      s2 = _lanesum(r, lane)
      ls = jnp.maximum(s2 + MARGIN, 0.0)
      return acc + ls

    return lax.fori_loop(0, C, tbody, total, unroll=2)

  idx_start(0, 0)
  idx_start(1, 1)
  idx_wait(0)
  g_start(0)

  def outer(k2, total):
    for b in range(2):
      k = 2 * k2 + b
      nb = 1 - b
      g_wait(b)

      @pl.when(k + 2 < ch)
      def _():
        idx_start(k + 2, b)

      @pl.when(k + 1 < ch)
      def _():
        idx_wait(nb)
        g_start(nb)

      total = compute(k, b, total)
    return total

  total = lax.fori_loop(0, ch // 2, outer, zero)
  acc_v[...] = total
  pltpu.sync_copy(acc_v, out.at[pl.ds(wid * L, L)])


def _reduce_body(n, x_ref, o_ref):
  o_ref[...] = (jnp.sum(x_ref[...]) / (L * n)).reshape(1, 1)


def kernel(fg_embed, triplet_index):
  n = triplet_index.shape[1]
  d = fg_embed.shape[1]
  ch = -(-n // (NW * C))
  ch = ch + (ch % 2)
  ch1 = (int(2 * ch * 0.39) // 2) * 2
  ch0 = 2 * ch - ch1
  npad = NS * (ch0 + ch1) * C

  idx = triplet_index.astype(jnp.int32)
  if npad > n:
    idx = jnp.pad(idx, ((0, 0), (0, npad - n)))

  mesh = plsc.VectorSubcoreMesh(core_axis_name="c", subcore_axis_name="s",
                                num_cores=NC, num_subcores=NS)
  sc_call = pl.kernel(
      functools.partial(_sc_body, n, ch0, ch1),
      out_type=jax.ShapeDtypeStruct((NW * L,), jnp.float32),
      mesh=mesh,
      scratch_types=(
          [pltpu.VMEM((C,), jnp.int32)] * 6
          + [pltpu.VMEM((C, d), jnp.float32)] * 6
          + [pltpu.VMEM((L,), jnp.float32)]
          + [pltpu.SemaphoreType.DMA] * 4
      ),
  )
  partials = sc_call(fg_embed, idx[0], idx[1], idx[2])

  loss = pl.pallas_call(
      functools.partial(_reduce_body, n),
      out_shape=jax.ShapeDtypeStruct((1, 1), jnp.float32),
  )(partials.reshape(4, NW * L // 4))
  return loss[0, 0]


# speedup vs baseline: 2.7643x; 1.0122x over previous
import functools

import jax
import jax.numpy as jnp
from jax import lax
from jax.experimental import pallas as pl
from jax.experimental.pallas import tpu as pltpu
from jax.experimental.pallas import tpu_sc as plsc

NC = 2
NS = 16
L = 16
NW = NC * NS
C = 160
MARGIN = 0.2

_GDN = lax.GatherDimensionNumbers(
    offset_dims=(), collapsed_slice_dims=(0,), start_index_map=(0,))


def _lanesum(v, lane):
  for sh in (8, 4, 2, 1):
    perm = lane ^ sh
    v = v + lax.gather(v, perm[:, None], _GDN, (1,),
                       mode=lax.GatherScatterMode.PROMISE_IN_BOUNDS)
  return v


def _sc_body(n, ch0, ch1, table, ia, ip, inn, out,
             i00, i01, i02, i10, i11, i12,
             r00, r01, r02, r10, r11, r12,
             acc_v, si0, si1, sr0, sr1):
  ib = ((i00, i01, i02), (i10, i11, i12))
  rb = ((r00, r01, r02), (r10, r11, r12))
  sib = (si0, si1)
  srb = (sr0, sr1)
  isrc = (ia, ip, inn)
  c = lax.axis_index("c")
  s = lax.axis_index("s")
  wid = s * NC + c
  ch = jnp.where(c == 0, ch0, ch1)
  wcbase = jnp.where(c == 0, s * ch0, NS * ch0 + s * ch1)
  lane = lax.iota(jnp.int32, 16)
  zero = jnp.zeros((L,), jnp.float32)

  def idx_start(k, b):
    base = (wcbase + k) * C
    for j in range(3):
      pltpu.async_copy(isrc[j].at[pl.ds(base, C)], ib[b][j], sib[b])

  def idx_wait(b):
    for j in range(3):
      pltpu.make_async_copy(isrc[j].at[pl.ds(0, C)], ib[b][j], sib[b]).wait()

  def g_start(b):
    for j in range(3):
      pltpu.async_copy(table.at[ib[b][j]], rb[b][j], srb[b])

  def g_wait(b):
    for j in range(3):
      pltpu.make_async_copy(table.at[ib[b][j]], rb[b][j], srb[b]).wait()

  def compute(k, b, total):
    base = (wcbase + k) * C
    ra, rp, rn = rb[b]
    d = table.shape[1]

    def tbody(t, acc):
      dap = zero
      dan = zero
      for kk in range(d // L):
        av = ra[t, pl.ds(kk * L, L)]
        pv = rp[t, pl.ds(kk * L, L)]
        nv = rn[t, pl.ds(kk * L, L)]
        dp = av - pv
        dn = av - nv
        dap = dap + dp * dp
        dan = dan + dn * dn
      r = dap - dan
      r = jnp.where(base + t < n, r, -1e30)
      s2 = _lanesum(r, lane)
      ls = jnp.maximum(s2 + MARGIN, 0.0)
      return acc + ls

    return lax.fori_loop(0, C, tbody, total, unroll=2)

  idx_start(0, 0)
  idx_start(1, 1)
  idx_wait(0)
  g_start(0)

  def outer(k2, total):
    for b in range(2):
      k = 2 * k2 + b
      nb = 1 - b
      g_wait(b)

      @pl.when(k + 2 < ch)
      def _():
        idx_start(k + 2, b)

      @pl.when(k + 1 < ch)
      def _():
        idx_wait(nb)
        g_start(nb)

      total = compute(k, b, total)
    return total

  total = lax.fori_loop(0, ch // 2, outer, zero)
  acc_v[...] = total
  pltpu.sync_copy(acc_v, out.at[pl.ds(wid * L, L)])


def _reduce_body(n, x_ref, o_ref):
  o_ref[...] = (jnp.sum(x_ref[...]) / (L * n)).reshape(1, 1)


def kernel(fg_embed, triplet_index):
  n = triplet_index.shape[1]
  d = fg_embed.shape[1]
  ch = -(-n // (NW * C))
  ch = ch + (ch % 2)
  ch1 = (int(2 * ch * 0.34) // 2) * 2
  ch0 = 2 * ch - ch1
  npad = NS * (ch0 + ch1) * C

  idx = triplet_index.astype(jnp.int32)
  if npad > n:
    idx = jnp.pad(idx, ((0, 0), (0, npad - n)))

  mesh = plsc.VectorSubcoreMesh(core_axis_name="c", subcore_axis_name="s",
                                num_cores=NC, num_subcores=NS)
  sc_call = pl.kernel(
      functools.partial(_sc_body, n, ch0, ch1),
      out_type=jax.ShapeDtypeStruct((NW * L,), jnp.float32),
      mesh=mesh,
      scratch_types=(
          [pltpu.VMEM((C,), jnp.int32)] * 6
          + [pltpu.VMEM((C, d), jnp.float32)] * 6
          + [pltpu.VMEM((L,), jnp.float32)]
          + [pltpu.SemaphoreType.DMA] * 4
      ),
  )
  partials = sc_call(fg_embed, idx[0], idx[1], idx[2])

  loss = pl.pallas_call(
      functools.partial(_reduce_body, n),
      out_shape=jax.ShapeDtypeStruct((1, 1), jnp.float32),
  )(partials.reshape(4, NW * L // 4))
  return loss[0, 0]
